# baseline (speedup 1.0000x reference)
import functools

import jax
import jax.numpy as jnp
import numpy as np
from jax import lax
from jax.experimental import pallas as pl
from jax.experimental.pallas import tpu as pltpu
from jax.experimental.pallas import tpu_sc as plsc

N = 10000
E = 320000
D = 128
G = 64

NC = 2
NS = 16
CH = 64
EPC = E // NC
EPT = EPC // NS
NCHUNK = EPT // CH
CH2 = EPT - NCHUNK * CH
ACC_ROWS = 10048
ZPT = ACC_ROWS // NS
RPT = 624


def _mm_body(x_ref, w_ref, o_ref):
    o_ref[...] = jnp.dot(x_ref[...], w_ref[...],
                         preferred_element_type=jnp.float32)


def _mm(x, w, bn):
    n, k = x.shape
    m = w.shape[1]
    return pl.pallas_call(
        _mm_body,
        grid=(n // bn,),
        in_specs=[pl.BlockSpec((bn, k), lambda i: (i, 0)),
                  pl.BlockSpec((k, m), lambda i: (0, 0))],
        out_specs=pl.BlockSpec((bn, m), lambda i: (i, 0)),
        out_shape=jax.ShapeDtypeStruct((n, m), jnp.float32),
    )(x, w)


def _mmb3_body(x_ref, w0_ref, b0_ref, w1_ref, b1_ref, w2_ref, b2_ref,
               o0_ref, o1_ref, o2_ref):
    x = x_ref[...]
    o0_ref[...] = (jnp.dot(x, w0_ref[...], preferred_element_type=jnp.float32)
                   + b0_ref[...]).astype(jnp.bfloat16)
    o1_ref[...] = (jnp.dot(x, w1_ref[...], preferred_element_type=jnp.float32)
                   + b1_ref[...]).astype(jnp.bfloat16)
    o2_ref[...] = (jnp.dot(x, w2_ref[...], preferred_element_type=jnp.float32)
                   + b2_ref[...]).astype(jnp.bfloat16)


def _mmb3(x, wb, bn):
    n, k = x.shape
    m = wb[0][0].shape[1]
    wspec = pl.BlockSpec((k, m), lambda i: (0, 0))
    bspec = pl.BlockSpec((1, m), lambda i: (0, 0))
    ospec = pl.BlockSpec((bn, m), lambda i: (i, 0))
    oshape = jax.ShapeDtypeStruct((n, m), jnp.bfloat16)
    return pl.pallas_call(
        _mmb3_body,
        grid=(n // bn,),
        in_specs=[pl.BlockSpec((bn, k), lambda i: (i, 0)),
                  wspec, bspec, wspec, bspec, wspec, bspec],
        out_specs=[ospec, ospec, ospec],
        out_shape=[oshape, oshape, oshape],
    )(x, wb[0][0], wb[0][1], wb[1][0], wb[1][1], wb[2][0], wb[2][1])


def _updf_body(a0_ref, a1_ref, h_ref, wa_ref, wh_ref, b_ref, wm_ref,
               o_ref, o2_ref):
    o = jnp.maximum(
        jnp.dot(a0_ref[...] + a1_ref[...], wa_ref[...],
                preferred_element_type=jnp.float32)
        + jnp.dot(h_ref[...], wh_ref[...], preferred_element_type=jnp.float32)
        + b_ref[...], 0.0)
    o_ref[...] = o
    o2_ref[...] = jnp.dot(o, wm_ref[...], preferred_element_type=jnp.float32)


def _updf(agg2, h, wa, wh, b, wm, bn):
    n, k = h.shape
    m = wa.shape[1]
    nb = n // bn
    wspec = pl.BlockSpec((k, m), lambda i: (0, 0))
    return pl.pallas_call(
        _updf_body,
        grid=(nb,),
        in_specs=[pl.BlockSpec((bn, k), lambda i: (i, 0)),
                  pl.BlockSpec((bn, k), lambda i: (i + nb, 0)),
                  pl.BlockSpec((bn, k), lambda i: (i, 0)),
                  wspec, wspec,
                  pl.BlockSpec((1, m), lambda i: (0, 0)),
                  wspec],
        out_specs=[pl.BlockSpec((bn, m), lambda i: (i, 0)),
                   pl.BlockSpec((bn, m), lambda i: (i, 0))],
        out_shape=[jax.ShapeDtypeStruct((n, m), jnp.float32),
                   jax.ShapeDtypeStruct((n, m), jnp.float32)],
    )(agg2, agg2, h, wa, wh, b, wm)


def _updpool_body(idx_ref, a0_ref, a1_ref, h_ref, wa_ref, wh_ref, b_ref,
                  o_ref):
    i = pl.program_id(0)
    y = jnp.maximum(
        jnp.dot(a0_ref[...] + a1_ref[...], wa_ref[...],
                preferred_element_type=jnp.float32)
        + jnp.dot(h_ref[...], wh_ref[...], preferred_element_type=jnp.float32)
        + b_ref[...], 0.0)
    idx = idx_ref[0]
    lab = lax.broadcasted_iota(jnp.int32, (G, idx.shape[1]), 0)
    onehot = (lab == idx).astype(jnp.float32)

    @pl.when(i == 0)
    def _():
        o_ref[...] = jnp.zeros_like(o_ref)

    o_ref[...] += jnp.dot(onehot, y, preferred_element_type=jnp.float32)


def _updpool(batch_idx3, agg2, h, wa, wh, b, bn):
    n, k = h.shape
    m = wa.shape[1]
    nb = n // bn
    wspec = pl.BlockSpec((k, m), lambda i: (0, 0))
    return pl.pallas_call(
        _updpool_body,
        grid=(nb,),
        in_specs=[pl.BlockSpec((1, 1, bn), lambda i: (i, 0, 0)),
                  pl.BlockSpec((bn, k), lambda i: (i, 0)),
                  pl.BlockSpec((bn, k), lambda i: (i + nb, 0)),
                  pl.BlockSpec((bn, k), lambda i: (i, 0)),
                  wspec, wspec,
                  pl.BlockSpec((1, m), lambda i: (0, 0))],
        out_specs=pl.BlockSpec((G, m), lambda i: (0, 0)),
        out_shape=jax.ShapeDtypeStruct((G, m), jnp.float32),
    )(batch_idx3, agg2, agg2, h, wa, wh, b)


def _fin_body(p_ref, w_ref, b_ref, o_ref):
    o_ref[...] = jnp.dot(p_ref[...], w_ref[...],
                         preferred_element_type=jnp.float32) + b_ref[...]


def _fin(pooled, w, b):
    return pl.pallas_call(
        _fin_body,
        in_specs=[pl.BlockSpec(pooled.shape, lambda: (0, 0)),
                  pl.BlockSpec(w.shape, lambda: (0, 0)),
                  pl.BlockSpec((1, 1), lambda: (0, 0))],
        out_specs=pl.BlockSpec((G, 1), lambda: (0, 0)),
        out_shape=jax.ShapeDtypeStruct((G, 1), jnp.float32),
    )(pooled, w, b)


NBUF = 3
NOUT = NCHUNK // NBUF
UNR = 8


def _sc_agg_body(a_hbm, b_hbm, id_hbm, out_hbm, acc,
                 iv0, iv1, iv2, dc0, dc1, dc2, dc3,
                 ar0, ar1, ar2, br0, br1,
                 ss0, ss1, ss2, sd0, sd1, sd2, sb0, sb1,
                 sg0, sg1, sg2, sx0, sx1, sx2):
    c = lax.axis_index("c")
    s = lax.axis_index("s")
    IV, DC = (iv0, iv1, iv2), (dc0, dc1, dc2)
    AR, BR = (ar0, ar1, ar2), (br0, br1)
    SS, SD, SB = (ss0, ss1, ss2), (sd0, sd1, sd2), (sb0, sb1)
    SG, SX = (sg0, sg1, sg2), (sx0, sx1, sx2)

    zero = jnp.zeros((16,), jnp.float32)

    def zrow_loop(r, carry):
        for k in range(D // 16):
            ar0[r, pl.ds(k * 16, 16)] = zero
        return carry
    lax.fori_loop(0, CH, zrow_loop, 0)

    def zloop64(r, carry):
        pltpu.sync_copy(ar0, acc.at[pl.ds(s * ZPT + r * 64, 64)])
        return carry
    lax.fori_loop(0, 9, zloop64, 0)

    def zloop8(r, carry):
        pltpu.sync_copy(ar0.at[pl.ds(0, 8)],
                        acc.at[pl.ds(s * ZPT + 576 + r * 8, 8)])
        return carry
    lax.fori_loop(0, 6, zloop8, 0)
    pltpu.sync_copy(ar0.at[pl.ds(0, 4)], acc.at[pl.ds(s * ZPT + 624, 4)])

    plsc.subcore_barrier()

    def ebase(g):
        return c * EPC + s * EPT + g * CH

    def copy_idx(g, p):
        pltpu.async_copy(id_hbm.at[pl.ds(ebase(g), CH)], IV[p].at[0], SS[p])
        pltpu.async_copy(id_hbm.at[pl.ds(E + ebase(g), CH)],
                         IV[p].at[1], SD[p])

    def copy_b(g, p2):
        pltpu.async_copy(b_hbm.at[pl.ds(ebase(g) * D, CH * D)],
                         BR[p2], SB[p2])

    def wait_idx(g, p):
        pltpu.make_async_copy(id_hbm.at[pl.ds(ebase(g), CH)],
                              IV[p].at[0], SS[p]).wait()

    def wait_dst(g, p):
        pltpu.make_async_copy(id_hbm.at[pl.ds(E + ebase(g), CH)],
                              IV[p].at[1], SD[p]).wait()

    def wait_scatter(p):
        pltpu.make_async_copy(AR[p], acc.at[DC[p]], SX[p]).wait()

    def issue_gather(p):
        pltpu.async_copy(a_hbm.at[IV[p].at[0]], AR[p], SG[p])

    def compute(p, p2, nrows):
        @plsc.parallel_loop(0, nrows, step=1, unroll=UNR)
        def _(j):
            for k in range(D // 32):
                blo, bhi = plsc.unpack(
                    BR[p2][pl.ds(j * D + k * 32, 32)],
                    format=plsc.PackFormat.INTERLEAVED,
                    preferred_element_type=jnp.float32)
                sl0 = pl.ds(k * 32, 16)
                sl1 = pl.ds(k * 32 + 16, 16)
                AR[p][j, sl0] = jnp.maximum(AR[p][j, sl0] + blo, 0.0)
                AR[p][j, sl1] = jnp.maximum(AR[p][j, sl1] + bhi, 0.0)

    def remap(g, p, nrows):
        wait_dst(g, p)
        for i in range(nrows // 16):
            sl = pl.ds(i * 16, 16)
            DC[p][sl] = IV[p][1, sl]

    copy_idx(0, 0)
    copy_idx(1, 1)
    copy_b(0, 0)
    wait_idx(0, 0)
    issue_gather(0)

    def outer_body(o, carry):
        for u in range(6):
            g = o * 6 + u
            p = u % NBUF
            p2 = u % 2
            q = (u + 1) % NBUF
            q2 = (u + 1) % 2
            r = (u + 2) % NBUF
            @pl.when(g + 1 <= NCHUNK - 1)
            def _():
                wait_idx(g + 1, q)

                @pl.when(g >= 2)
                def _():
                    wait_scatter(q)
                issue_gather(q)
                copy_b(g + 1, q2)
            @pl.when(g + 2 <= NCHUNK - 1)
            def _():
                copy_idx(g + 2, r)
            pltpu.make_async_copy(a_hbm.at[IV[p].at[0]], AR[p], SG[p]).wait()
            pltpu.make_async_copy(b_hbm.at[pl.ds(ebase(g) * D, CH * D)],
                                  BR[p2], SB[p2]).wait()
            compute(p, p2, CH)
            remap(g, p, CH)
            pltpu.async_copy(AR[p], acc.at[DC[p]], SX[p], add=True)
        return carry
    lax.fori_loop(0, NCHUNK // 6, outer_body, 0)

    for p in range(NBUF):
        wait_scatter(p)

    rbase = c * EPC + s * EPT + NCHUNK * CH
    pltpu.sync_copy(id_hbm.at[pl.ds(rbase, CH2)], iv0.at[0, pl.ds(0, CH2)])
    pltpu.sync_copy(id_hbm.at[pl.ds(E + rbase, CH2)],
                    iv0.at[1, pl.ds(0, CH2)])
    pltpu.sync_copy(b_hbm.at[pl.ds(rbase * D, CH2 * D)],
                    br0.at[pl.ds(0, CH2 * D)])
    pltpu.async_copy(a_hbm.at[iv0.at[0, pl.ds(0, CH2)]],
                     ar0.at[pl.ds(0, CH2)], sg0).wait()
    compute(0, 0, CH2)
    for i in range(CH2 // 16):
        sl = pl.ds(i * 16, 16)
        dc3[sl] = iv0[1, sl]
    pltpu.sync_copy(ar0.at[pl.ds(0, CH2)], acc.at[dc3], add=True)

    plsc.subcore_barrier()

    out0 = c * N + s * RPT
    pltpu.sync_copy(acc.at[pl.ds(s * RPT, RPT)], out_hbm.at[pl.ds(out0, RPT)])

    @pl.when(s == 0)
    def _():
        pltpu.sync_copy(acc.at[pl.ds(NS * RPT, N - NS * RPT)],
                        out_hbm.at[pl.ds(c * N + NS * RPT, N - NS * RPT)])


def _sc_agg(a, b, id_xe):
    mesh = plsc.VectorSubcoreMesh(core_axis_name="c", subcore_axis_name="s")
    f = functools.partial(
        pl.kernel,
        mesh=mesh,
        compiler_params=pltpu.CompilerParams(
            needs_layout_passes=False, use_tc_tiling_on_sc=False),
        out_type=jax.ShapeDtypeStruct((NC * N, D), jnp.float32),
        scratch_types=(
            [pltpu.VMEM_SHARED((ACC_ROWS, D), jnp.float32)]
            + [pltpu.VMEM((2, CH), jnp.int32)] * 3
            + [pltpu.VMEM((CH,), jnp.int32)] * 3
            + [pltpu.VMEM((CH2,), jnp.int32)]
            + [pltpu.VMEM((CH, D), jnp.float32)] * 3
            + [pltpu.VMEM((CH * D,), jnp.bfloat16)] * 2
            + [pltpu.SemaphoreType.DMA] * 14
        ),
    )(_sc_agg_body)
    return f(a, b, id_xe)


_BPERM = np.concatenate(
    [np.stack([np.arange(k * 32, k * 32 + 16),
               np.arange(k * 32 + 16, k * 32 + 32)], 1).reshape(-1)
     for k in range(D // 32)])


def kernel(H, Xe, id_Xe, batch_idx, Wm0, bm0, Wu0, bu0, Wm1, bm1, Wu1, bu1,
           Wm2, bm2, Wu2, bu2, Wmlp, bmlp):
    b0, b1, b2 = _mmb3(Xe, ((Wm0[D:][:, _BPERM], bm0[_BPERM].reshape(1, -1)),
                            (Wm1[D:][:, _BPERM], bm1[_BPERM].reshape(1, -1)),
                            (Wm2[D:][:, _BPERM], bm2[_BPERM].reshape(1, -1))),
                       3200)
    a0 = _mm(H, Wm0[:D], 2000)
    id_flat = id_Xe.reshape(-1)
    agg0 = _sc_agg(a0, b0.reshape(-1), id_flat)
    h1, a1 = _updf(agg0, H, Wu0[:D], Wu0[D:], bu0.reshape(1, -1),
                   Wm1[:D], 2000)
    agg1 = _sc_agg(a1, b1.reshape(-1), id_flat)
    h2, a2 = _updf(agg1, h1, Wu1[:D], Wu1[D:], bu1.reshape(1, -1),
                   Wm2[:D], 2000)
    agg2 = _sc_agg(a2, b2.reshape(-1), id_flat)
    pooled = _updpool(batch_idx.reshape(N // 1000, 1, 1000), agg2, h2,
                      Wu2[:D], Wu2[D:], bu2.reshape(1, -1), 1000)
    return _fin(pooled, Wmlp, bmlp.reshape(1, 1))


# speedup vs baseline: 1.4784x; 1.4784x over previous
import functools

import jax
import jax.numpy as jnp
import numpy as np
from jax import lax
from jax.experimental import pallas as pl
from jax.experimental.pallas import tpu as pltpu
from jax.experimental.pallas import tpu_sc as plsc

N = 10000
E = 320000
D = 128
G = 64

NC = 2
NS = 16
CH = 64
EPC = E // NC
EPT = EPC // NS
NCHUNK = EPT // CH
CH2 = EPT - NCHUNK * CH
ACC_ROWS = 10048
ZPT = ACC_ROWS // NS
RPT = 624


def _mm_body(x_ref, w_ref, o_ref):
    o_ref[...] = jnp.dot(x_ref[...], w_ref[...],
                         preferred_element_type=jnp.float32)


def _mm(x, w, bn):
    n, k = x.shape
    m = w.shape[1]
    return pl.pallas_call(
        _mm_body,
        grid=(n // bn,),
        in_specs=[pl.BlockSpec((bn, k), lambda i: (i, 0)),
                  pl.BlockSpec((k, m), lambda i: (0, 0))],
        out_specs=pl.BlockSpec((bn, m), lambda i: (i, 0)),
        out_shape=jax.ShapeDtypeStruct((n, m), jnp.float32),
    )(x, w)


def _mmb3_body(x_ref, w0_ref, b0_ref, w1_ref, b1_ref, w2_ref, b2_ref,
               o0_ref, o1_ref, o2_ref):
    x = x_ref[...]
    o0_ref[...] = (jnp.dot(x, w0_ref[...], preferred_element_type=jnp.float32)
                   + b0_ref[...]).astype(jnp.bfloat16)
    o1_ref[...] = (jnp.dot(x, w1_ref[...], preferred_element_type=jnp.float32)
                   + b1_ref[...]).astype(jnp.bfloat16)
    o2_ref[...] = (jnp.dot(x, w2_ref[...], preferred_element_type=jnp.float32)
                   + b2_ref[...]).astype(jnp.bfloat16)


def _mmb3(x, wb, bn):
    n, k = x.shape
    m = wb[0][0].shape[1]
    wspec = pl.BlockSpec((k, m), lambda i: (0, 0))
    bspec = pl.BlockSpec((1, m), lambda i: (0, 0))
    ospec = pl.BlockSpec((bn, m), lambda i: (i, 0))
    oshape = jax.ShapeDtypeStruct((n, m), jnp.bfloat16)
    return pl.pallas_call(
        _mmb3_body,
        grid=(n // bn,),
        in_specs=[pl.BlockSpec((bn, k), lambda i: (i, 0)),
                  wspec, bspec, wspec, bspec, wspec, bspec],
        out_specs=[ospec, ospec, ospec],
        out_shape=[oshape, oshape, oshape],
    )(x, wb[0][0], wb[0][1], wb[1][0], wb[1][1], wb[2][0], wb[2][1])


def _updf_body(a0_ref, a1_ref, h_ref, wa_ref, wh_ref, b_ref, wm_ref,
               o_ref, o2_ref):
    o = jnp.maximum(
        jnp.dot(a0_ref[...] + a1_ref[...], wa_ref[...],
                preferred_element_type=jnp.float32)
        + jnp.dot(h_ref[...], wh_ref[...], preferred_element_type=jnp.float32)
        + b_ref[...], 0.0)
    o_ref[...] = o
    o2_ref[...] = jnp.dot(o, wm_ref[...], preferred_element_type=jnp.float32)


def _updf(agg2, h, wa, wh, b, wm, bn):
    n, k = h.shape
    m = wa.shape[1]
    nb = n // bn
    wspec = pl.BlockSpec((k, m), lambda i: (0, 0))
    return pl.pallas_call(
        _updf_body,
        grid=(nb,),
        in_specs=[pl.BlockSpec((bn, k), lambda i: (i, 0)),
                  pl.BlockSpec((bn, k), lambda i: (i + nb, 0)),
                  pl.BlockSpec((bn, k), lambda i: (i, 0)),
                  wspec, wspec,
                  pl.BlockSpec((1, m), lambda i: (0, 0)),
                  wspec],
        out_specs=[pl.BlockSpec((bn, m), lambda i: (i, 0)),
                   pl.BlockSpec((bn, m), lambda i: (i, 0))],
        out_shape=[jax.ShapeDtypeStruct((n, m), jnp.float32),
                   jax.ShapeDtypeStruct((n, m), jnp.float32)],
    )(agg2, agg2, h, wa, wh, b, wm)


def _updpool_body(idx_ref, a0_ref, a1_ref, h_ref, wa_ref, wh_ref, b_ref,
                  o_ref):
    i = pl.program_id(0)
    y = jnp.maximum(
        jnp.dot(a0_ref[...] + a1_ref[...], wa_ref[...],
                preferred_element_type=jnp.float32)
        + jnp.dot(h_ref[...], wh_ref[...], preferred_element_type=jnp.float32)
        + b_ref[...], 0.0)
    idx = idx_ref[0]
    lab = lax.broadcasted_iota(jnp.int32, (G, idx.shape[1]), 0)
    onehot = (lab == idx).astype(jnp.float32)

    @pl.when(i == 0)
    def _():
        o_ref[...] = jnp.zeros_like(o_ref)

    o_ref[...] += jnp.dot(onehot, y, preferred_element_type=jnp.float32)


def _updpool(batch_idx3, agg2, h, wa, wh, b, bn):
    n, k = h.shape
    m = wa.shape[1]
    nb = n // bn
    wspec = pl.BlockSpec((k, m), lambda i: (0, 0))
    return pl.pallas_call(
        _updpool_body,
        grid=(nb,),
        in_specs=[pl.BlockSpec((1, 1, bn), lambda i: (i, 0, 0)),
                  pl.BlockSpec((bn, k), lambda i: (i, 0)),
                  pl.BlockSpec((bn, k), lambda i: (i + nb, 0)),
                  pl.BlockSpec((bn, k), lambda i: (i, 0)),
                  wspec, wspec,
                  pl.BlockSpec((1, m), lambda i: (0, 0))],
        out_specs=pl.BlockSpec((G, m), lambda i: (0, 0)),
        out_shape=jax.ShapeDtypeStruct((G, m), jnp.float32),
    )(batch_idx3, agg2, agg2, h, wa, wh, b)


def _fin_body(p_ref, w_ref, b_ref, o_ref):
    o_ref[...] = jnp.dot(p_ref[...], w_ref[...],
                         preferred_element_type=jnp.float32) + b_ref[...]


def _fin(pooled, w, b):
    return pl.pallas_call(
        _fin_body,
        in_specs=[pl.BlockSpec(pooled.shape, lambda: (0, 0)),
                  pl.BlockSpec(w.shape, lambda: (0, 0)),
                  pl.BlockSpec((1, 1), lambda: (0, 0))],
        out_specs=pl.BlockSpec((G, 1), lambda: (0, 0)),
        out_shape=jax.ShapeDtypeStruct((G, 1), jnp.float32),
    )(pooled, w, b)


NBUF = 3
NOUT = NCHUNK // NBUF
UNR = 8


def _sc_agg_body(a_hbm, b_hbm, id_hbm, out_hbm, acc,
                 iv0, iv1, iv2, dc0, dc1, dc2, dc3,
                 ar0, ar1, ar2, br0, br1,
                 ss0, ss1, ss2, sd0, sd1, sd2, sb0, sb1,
                 sg0, sg1, sg2, sx0, sx1, sx2):
    c = lax.axis_index("c")
    s = lax.axis_index("s")
    IV, DC = (iv0, iv1, iv2), (dc0, dc1, dc2)
    AR, BR = (ar0, ar1, ar2), (br0, br1)
    SS, SD, SB = (ss0, ss1, ss2), (sd0, sd1, sd2), (sb0, sb1)
    SG, SX = (sg0, sg1, sg2), (sx0, sx1, sx2)

    zero = jnp.zeros((16,), jnp.float32)

    def zrow_loop(r, carry):
        for k in range(D // 16):
            ar0[r, pl.ds(k * 16, 16)] = zero
        return carry
    lax.fori_loop(0, CH, zrow_loop, 0)

    def zloop64(r, carry):
        pltpu.sync_copy(ar0, acc.at[pl.ds(s * ZPT + r * 64, 64)])
        return carry
    lax.fori_loop(0, 9, zloop64, 0)

    def zloop8(r, carry):
        pltpu.sync_copy(ar0.at[pl.ds(0, 8)],
                        acc.at[pl.ds(s * ZPT + 576 + r * 8, 8)])
        return carry
    lax.fori_loop(0, 6, zloop8, 0)
    pltpu.sync_copy(ar0.at[pl.ds(0, 4)], acc.at[pl.ds(s * ZPT + 624, 4)])

    plsc.subcore_barrier()

    def ebase(g):
        return c * EPC + s * EPT + g * CH

    def copy_idx(g, p):
        pltpu.async_copy(id_hbm.at[pl.ds(ebase(g), CH)], IV[p].at[0], SS[p])
        pltpu.async_copy(id_hbm.at[pl.ds(E + ebase(g), CH)],
                         IV[p].at[1], SD[p])

    def copy_b(g, p2):
        pltpu.async_copy(b_hbm.at[pl.ds(ebase(g) // 2, CH // 2)],
                         BR[p2], SB[p2])

    def wait_idx(g, p):
        pltpu.make_async_copy(id_hbm.at[pl.ds(ebase(g), CH)],
                              IV[p].at[0], SS[p]).wait()

    def wait_dst(g, p):
        pltpu.make_async_copy(id_hbm.at[pl.ds(E + ebase(g), CH)],
                              IV[p].at[1], SD[p]).wait()

    def wait_scatter(p):
        pltpu.make_async_copy(AR[p], acc.at[DC[p]], SX[p]).wait()

    def issue_gather(p):
        pltpu.async_copy(a_hbm.at[IV[p].at[0]], AR[p], SG[p])

    def compute(p, p2, nrows):
        @plsc.parallel_loop(0, nrows // 2, step=1, unroll=UNR // 2)
        def _(j2):
            for u in range(2):
                j = j2 * 2 + u
                for k in range(D // 32):
                    v = BR[p2][j2, u, pl.ds(k * 32, 32)].astype(jnp.float32)
                    sl0 = pl.ds(k * 32, 16)
                    sl1 = pl.ds(k * 32 + 16, 16)
                    AR[p][j, sl0] = jnp.maximum(
                        AR[p][j, sl0] + lax.slice(v, (0,), (16,)), 0.0)
                    AR[p][j, sl1] = jnp.maximum(
                        AR[p][j, sl1] + lax.slice(v, (16,), (32,)), 0.0)

    def remap(g, p, nrows):
        wait_dst(g, p)
        for i in range(nrows // 16):
            sl = pl.ds(i * 16, 16)
            DC[p][sl] = IV[p][1, sl]

    copy_idx(0, 0)
    copy_idx(1, 1)
    copy_b(0, 0)
    wait_idx(0, 0)
    issue_gather(0)

    def outer_body(o, carry):
        for u in range(6):
            g = o * 6 + u
            p = u % NBUF
            p2 = u % 2
            q = (u + 1) % NBUF
            q2 = (u + 1) % 2
            r = (u + 2) % NBUF
            @pl.when(g + 1 <= NCHUNK - 1)
            def _():
                wait_idx(g + 1, q)

                @pl.when(g >= 2)
                def _():
                    wait_scatter(q)
                issue_gather(q)
                copy_b(g + 1, q2)
            @pl.when(g + 2 <= NCHUNK - 1)
            def _():
                copy_idx(g + 2, r)
            pltpu.make_async_copy(a_hbm.at[IV[p].at[0]], AR[p], SG[p]).wait()
            pltpu.make_async_copy(b_hbm.at[pl.ds(ebase(g) // 2, CH // 2)],
                                  BR[p2], SB[p2]).wait()
            compute(p, p2, CH)
            remap(g, p, CH)
            pltpu.async_copy(AR[p], acc.at[DC[p]], SX[p], add=True)
        return carry
    lax.fori_loop(0, NCHUNK // 6, outer_body, 0)

    for p in range(NBUF):
        wait_scatter(p)

    rbase = c * EPC + s * EPT + NCHUNK * CH
    pltpu.sync_copy(id_hbm.at[pl.ds(rbase, CH2)], iv0.at[0, pl.ds(0, CH2)])
    pltpu.sync_copy(id_hbm.at[pl.ds(E + rbase, CH2)],
                    iv0.at[1, pl.ds(0, CH2)])
    pltpu.sync_copy(b_hbm.at[pl.ds(rbase // 2, CH2 // 2)],
                    br0.at[pl.ds(0, CH2 // 2)])
    pltpu.async_copy(a_hbm.at[iv0.at[0, pl.ds(0, CH2)]],
                     ar0.at[pl.ds(0, CH2)], sg0).wait()
    compute(0, 0, CH2)
    for i in range(CH2 // 16):
        sl = pl.ds(i * 16, 16)
        dc3[sl] = iv0[1, sl]
    pltpu.sync_copy(ar0.at[pl.ds(0, CH2)], acc.at[dc3], add=True)

    plsc.subcore_barrier()

    out0 = c * N + s * RPT
    pltpu.sync_copy(acc.at[pl.ds(s * RPT, RPT)], out_hbm.at[pl.ds(out0, RPT)])

    @pl.when(s == 0)
    def _():
        pltpu.sync_copy(acc.at[pl.ds(NS * RPT, N - NS * RPT)],
                        out_hbm.at[pl.ds(c * N + NS * RPT, N - NS * RPT)])


def _sc_agg(a, b, id_xe):
    mesh = plsc.VectorSubcoreMesh(core_axis_name="c", subcore_axis_name="s")
    f = functools.partial(
        pl.kernel,
        mesh=mesh,
        out_type=jax.ShapeDtypeStruct((NC * N, D), jnp.float32),
        scratch_types=(
            [pltpu.VMEM_SHARED((ACC_ROWS, D), jnp.float32)]
            + [pltpu.VMEM((2, CH), jnp.int32)] * 3
            + [pltpu.VMEM((CH,), jnp.int32)] * 3
            + [pltpu.VMEM((CH2,), jnp.int32)]
            + [pltpu.VMEM((CH, D), jnp.float32)] * 3
            + [pltpu.VMEM((CH // 2, 2, D), jnp.bfloat16)] * 2
            + [pltpu.SemaphoreType.DMA] * 14
        ),
    )(_sc_agg_body)
    return f(a, b, id_xe)


def kernel(H, Xe, id_Xe, batch_idx, Wm0, bm0, Wu0, bu0, Wm1, bm1, Wu1, bu1,
           Wm2, bm2, Wu2, bu2, Wmlp, bmlp):
    b0, b1, b2 = _mmb3(Xe, ((Wm0[D:], bm0.reshape(1, -1)),
                            (Wm1[D:], bm1.reshape(1, -1)),
                            (Wm2[D:], bm2.reshape(1, -1))), 3200)
    a0 = _mm(H, Wm0[:D], 2000)
    id_flat = id_Xe.reshape(-1)
    agg0 = _sc_agg(a0, b0.reshape(E // 2, 2, D), id_flat)
    h1, a1 = _updf(agg0, H, Wu0[:D], Wu0[D:], bu0.reshape(1, -1),
                   Wm1[:D], 2000)
    agg1 = _sc_agg(a1, b1.reshape(E // 2, 2, D), id_flat)
    h2, a2 = _updf(agg1, h1, Wu1[:D], Wu1[D:], bu1.reshape(1, -1),
                   Wm2[:D], 2000)
    agg2 = _sc_agg(a2, b2.reshape(E // 2, 2, D), id_flat)
    pooled = _updpool(batch_idx.reshape(N // 1000, 1, 1000), agg2, h2,
                      Wu2[:D], Wu2[D:], bu2.reshape(1, -1), 1000)
    return _fin(pooled, Wmlp, bmlp.reshape(1, 1))


# speedup vs baseline: 1.5707x; 1.0624x over previous
import functools

import jax
import jax.numpy as jnp
import numpy as np
from jax import lax
from jax.experimental import pallas as pl
from jax.experimental.pallas import tpu as pltpu
from jax.experimental.pallas import tpu_sc as plsc

N = 10000
E = 320000
D = 128
G = 64

NC = 2
NS = 16
CH = 80
EPC = E // NC
EPT = EPC // NS
NCHUNK = EPT // CH
ACC_ROWS = 10048
ZPT = ACC_ROWS // NS
RPT = 624


def _mm_body(x_ref, w_ref, o_ref):
    o_ref[...] = jnp.dot(x_ref[...], w_ref[...],
                         preferred_element_type=jnp.float32)


def _mm(x, w, bn):
    n, k = x.shape
    m = w.shape[1]
    return pl.pallas_call(
        _mm_body,
        grid=(n // bn,),
        in_specs=[pl.BlockSpec((bn, k), lambda i: (i, 0)),
                  pl.BlockSpec((k, m), lambda i: (0, 0))],
        out_specs=pl.BlockSpec((bn, m), lambda i: (i, 0)),
        out_shape=jax.ShapeDtypeStruct((n, m), jnp.float32),
    )(x, w)


def _mmb3_body(x_ref, w0_ref, b0_ref, w1_ref, b1_ref, w2_ref, b2_ref,
               o0_ref, o1_ref, o2_ref):
    x = x_ref[...]
    o0_ref[...] = (jnp.dot(x, w0_ref[...], preferred_element_type=jnp.float32)
                   + b0_ref[...]).astype(jnp.bfloat16)
    o1_ref[...] = (jnp.dot(x, w1_ref[...], preferred_element_type=jnp.float32)
                   + b1_ref[...]).astype(jnp.bfloat16)
    o2_ref[...] = (jnp.dot(x, w2_ref[...], preferred_element_type=jnp.float32)
                   + b2_ref[...]).astype(jnp.bfloat16)


def _mmb3(x, wb, bn):
    n, k = x.shape
    m = wb[0][0].shape[1]
    wspec = pl.BlockSpec((k, m), lambda i: (0, 0))
    bspec = pl.BlockSpec((1, m), lambda i: (0, 0))
    ospec = pl.BlockSpec((bn, m), lambda i: (i, 0))
    oshape = jax.ShapeDtypeStruct((n, m), jnp.bfloat16)
    return pl.pallas_call(
        _mmb3_body,
        grid=(n // bn,),
        in_specs=[pl.BlockSpec((bn, k), lambda i: (i, 0)),
                  wspec, bspec, wspec, bspec, wspec, bspec],
        out_specs=[ospec, ospec, ospec],
        out_shape=[oshape, oshape, oshape],
    )(x, wb[0][0], wb[0][1], wb[1][0], wb[1][1], wb[2][0], wb[2][1])


def _updf_body(a0_ref, a1_ref, h_ref, wa_ref, wh_ref, b_ref, wm_ref,
               o_ref, o2_ref):
    o = jnp.maximum(
        jnp.dot(a0_ref[...] + a1_ref[...], wa_ref[...],
                preferred_element_type=jnp.float32)
        + jnp.dot(h_ref[...], wh_ref[...], preferred_element_type=jnp.float32)
        + b_ref[...], 0.0)
    o_ref[...] = o
    o2_ref[...] = jnp.dot(o, wm_ref[...], preferred_element_type=jnp.float32)


def _updf(agg2, h, wa, wh, b, wm, bn):
    n, k = h.shape
    m = wa.shape[1]
    nb = n // bn
    wspec = pl.BlockSpec((k, m), lambda i: (0, 0))
    return pl.pallas_call(
        _updf_body,
        grid=(nb,),
        in_specs=[pl.BlockSpec((bn, k), lambda i: (i, 0)),
                  pl.BlockSpec((bn, k), lambda i: (i + nb, 0)),
                  pl.BlockSpec((bn, k), lambda i: (i, 0)),
                  wspec, wspec,
                  pl.BlockSpec((1, m), lambda i: (0, 0)),
                  wspec],
        out_specs=[pl.BlockSpec((bn, m), lambda i: (i, 0)),
                   pl.BlockSpec((bn, m), lambda i: (i, 0))],
        out_shape=[jax.ShapeDtypeStruct((n, m), jnp.float32),
                   jax.ShapeDtypeStruct((n, m), jnp.float32)],
    )(agg2, agg2, h, wa, wh, b, wm)


def _updpool_body(idx_ref, a0_ref, a1_ref, h_ref, wa_ref, wh_ref, b_ref,
                  o_ref):
    i = pl.program_id(0)
    y = jnp.maximum(
        jnp.dot(a0_ref[...] + a1_ref[...], wa_ref[...],
                preferred_element_type=jnp.float32)
        + jnp.dot(h_ref[...], wh_ref[...], preferred_element_type=jnp.float32)
        + b_ref[...], 0.0)
    idx = idx_ref[0]
    lab = lax.broadcasted_iota(jnp.int32, (G, idx.shape[1]), 0)
    onehot = (lab == idx).astype(jnp.float32)

    @pl.when(i == 0)
    def _():
        o_ref[...] = jnp.zeros_like(o_ref)

    o_ref[...] += jnp.dot(onehot, y, preferred_element_type=jnp.float32)


def _updpool(batch_idx3, agg2, h, wa, wh, b, bn):
    n, k = h.shape
    m = wa.shape[1]
    nb = n // bn
    wspec = pl.BlockSpec((k, m), lambda i: (0, 0))
    return pl.pallas_call(
        _updpool_body,
        grid=(nb,),
        in_specs=[pl.BlockSpec((1, 1, bn), lambda i: (i, 0, 0)),
                  pl.BlockSpec((bn, k), lambda i: (i, 0)),
                  pl.BlockSpec((bn, k), lambda i: (i + nb, 0)),
                  pl.BlockSpec((bn, k), lambda i: (i, 0)),
                  wspec, wspec,
                  pl.BlockSpec((1, m), lambda i: (0, 0))],
        out_specs=pl.BlockSpec((G, m), lambda i: (0, 0)),
        out_shape=jax.ShapeDtypeStruct((G, m), jnp.float32),
    )(batch_idx3, agg2, agg2, h, wa, wh, b)


def _fin_body(p_ref, w_ref, b_ref, o_ref):
    o_ref[...] = jnp.dot(p_ref[...], w_ref[...],
                         preferred_element_type=jnp.float32) + b_ref[...]


def _fin(pooled, w, b):
    return pl.pallas_call(
        _fin_body,
        in_specs=[pl.BlockSpec(pooled.shape, lambda: (0, 0)),
                  pl.BlockSpec(w.shape, lambda: (0, 0)),
                  pl.BlockSpec((1, 1), lambda: (0, 0))],
        out_specs=pl.BlockSpec((G, 1), lambda: (0, 0)),
        out_shape=jax.ShapeDtypeStruct((G, 1), jnp.float32),
    )(pooled, w, b)


NBUF = 3
NMAIN = 120
UNR = 8


def _sc_agg_body(a_hbm, b_hbm, id_hbm, out_hbm, acc,
                 iv0, iv1, iv2, dc0, dc1, dc2,
                 ar0, ar1, ar2, br0, br1,
                 ss0, ss1, ss2, sd0, sd1, sd2, sb0, sb1,
                 sg0, sg1, sg2, sx0, sx1, sx2):
    c = lax.axis_index("c")
    s = lax.axis_index("s")
    IV, DC = (iv0, iv1, iv2), (dc0, dc1, dc2)
    AR, BR = (ar0, ar1, ar2), (br0, br1)
    SS, SD, SB = (ss0, ss1, ss2), (sd0, sd1, sd2), (sb0, sb1)
    SG, SX = (sg0, sg1, sg2), (sx0, sx1, sx2)

    zero = jnp.zeros((16,), jnp.float32)

    def zrow_loop(r, carry):
        for k in range(D // 16):
            ar0[r, pl.ds(k * 16, 16)] = zero
        return carry
    lax.fori_loop(0, CH, zrow_loop, 0)

    def zloop64(r, carry):
        pltpu.sync_copy(ar0.at[pl.ds(0, 64)],
                        acc.at[pl.ds(s * ZPT + r * 64, 64)])
        return carry
    lax.fori_loop(0, 9, zloop64, 0)

    def zloop8(r, carry):
        pltpu.sync_copy(ar0.at[pl.ds(0, 8)],
                        acc.at[pl.ds(s * ZPT + 576 + r * 8, 8)])
        return carry
    lax.fori_loop(0, 6, zloop8, 0)
    pltpu.sync_copy(ar0.at[pl.ds(0, 4)], acc.at[pl.ds(s * ZPT + 624, 4)])

    plsc.subcore_barrier()

    def ebase(g):
        return c * EPC + s * EPT + g * CH

    def copy_idx(g, p):
        pltpu.async_copy(id_hbm.at[pl.ds(ebase(g), CH)], IV[p].at[0], SS[p])
        pltpu.async_copy(id_hbm.at[pl.ds(E + ebase(g), CH)],
                         IV[p].at[1], SD[p])

    def copy_b(g, p2):
        pltpu.async_copy(b_hbm.at[pl.ds(ebase(g) // 2, CH // 2)],
                         BR[p2], SB[p2])

    def wait_idx(g, p):
        pltpu.make_async_copy(id_hbm.at[pl.ds(ebase(g), CH)],
                              IV[p].at[0], SS[p]).wait()

    def wait_dst(g, p):
        pltpu.make_async_copy(id_hbm.at[pl.ds(E + ebase(g), CH)],
                              IV[p].at[1], SD[p]).wait()

    def wait_scatter(p):
        pltpu.make_async_copy(AR[p], acc.at[DC[p]], SX[p]).wait()

    def issue_gather(p):
        pltpu.async_copy(a_hbm.at[IV[p].at[0]], AR[p], SG[p])

    def compute(p, p2, nrows):
        @plsc.parallel_loop(0, nrows // 2, step=1, unroll=UNR // 2)
        def _(j2):
            for u in range(2):
                j = j2 * 2 + u
                for k in range(D // 32):
                    v = BR[p2][j2, u, pl.ds(k * 32, 32)].astype(jnp.float32)
                    sl0 = pl.ds(k * 32, 16)
                    sl1 = pl.ds(k * 32 + 16, 16)
                    AR[p][j, sl0] = jnp.maximum(
                        AR[p][j, sl0] + lax.slice(v, (0,), (16,)), 0.0)
                    AR[p][j, sl1] = jnp.maximum(
                        AR[p][j, sl1] + lax.slice(v, (16,), (32,)), 0.0)

    def remap(g, p, nrows):
        wait_dst(g, p)
        for i in range(nrows // 16):
            sl = pl.ds(i * 16, 16)
            DC[p][sl] = IV[p][1, sl]

    copy_idx(0, 0)
    copy_idx(1, 1)
    copy_b(0, 0)
    wait_idx(0, 0)
    issue_gather(0)

    def chunk_body(g, p, p2, q, q2, r, prefetch1, prefetch2, guard):
        if prefetch1:
            wait_idx(g + 1, q)
            if guard:
                @pl.when(g >= 2)
                def _():
                    wait_scatter(q)
            else:
                wait_scatter(q)
            issue_gather(q)
            copy_b(g + 1, q2)
        if prefetch2:
            copy_idx(g + 2, r)
        pltpu.make_async_copy(a_hbm.at[IV[p].at[0]], AR[p], SG[p]).wait()
        pltpu.make_async_copy(b_hbm.at[pl.ds(ebase(g) // 2, CH // 2)],
                              BR[p2], SB[p2]).wait()
        compute(p, p2, CH)
        remap(g, p, CH)
        pltpu.async_copy(AR[p], acc.at[DC[p]], SX[p], add=True)

    def outer_body(o, carry):
        for u in range(6):
            chunk_body(o * 6 + u, u % NBUF, u % 2, (u + 1) % NBUF,
                       (u + 1) % 2, (u + 2) % NBUF, True, True, True)
        return carry
    lax.fori_loop(0, NMAIN // 6, outer_body, 0)

    for g in range(NMAIN, NCHUNK):
        chunk_body(g, g % NBUF, g % 2, (g + 1) % NBUF, (g + 1) % 2,
                   (g + 2) % NBUF, g + 1 < NCHUNK, g + 2 < NCHUNK, False)

    for p in range(NBUF):
        wait_scatter(p)

    plsc.subcore_barrier()

    out0 = c * N + s * RPT
    pltpu.sync_copy(acc.at[pl.ds(s * RPT, RPT)], out_hbm.at[pl.ds(out0, RPT)])

    @pl.when(s == 0)
    def _():
        pltpu.sync_copy(acc.at[pl.ds(NS * RPT, N - NS * RPT)],
                        out_hbm.at[pl.ds(c * N + NS * RPT, N - NS * RPT)])


def _sc_agg(a, b, id_xe):
    mesh = plsc.VectorSubcoreMesh(core_axis_name="c", subcore_axis_name="s")
    f = functools.partial(
        pl.kernel,
        mesh=mesh,
        out_type=jax.ShapeDtypeStruct((NC * N, D), jnp.float32),
        scratch_types=(
            [pltpu.VMEM_SHARED((ACC_ROWS, D), jnp.float32)]
            + [pltpu.VMEM((2, CH), jnp.int32)] * 3
            + [pltpu.VMEM((CH,), jnp.int32)] * 3
            + [pltpu.VMEM((CH, D), jnp.float32)] * 3
            + [pltpu.VMEM((CH // 2, 2, D), jnp.bfloat16)] * 2
            + [pltpu.SemaphoreType.DMA] * 14
        ),
    )(_sc_agg_body)
    return f(a, b, id_xe)


def kernel(H, Xe, id_Xe, batch_idx, Wm0, bm0, Wu0, bu0, Wm1, bm1, Wu1, bu1,
           Wm2, bm2, Wu2, bu2, Wmlp, bmlp):
    b0, b1, b2 = _mmb3(Xe, ((Wm0[D:], bm0.reshape(1, -1)),
                            (Wm1[D:], bm1.reshape(1, -1)),
                            (Wm2[D:], bm2.reshape(1, -1))), 3200)
    a0 = _mm(H, Wm0[:D], 2000)
    id_flat = id_Xe.reshape(-1)
    agg0 = _sc_agg(a0, b0.reshape(E // 2, 2, D), id_flat)
    h1, a1 = _updf(agg0, H, Wu0[:D], Wu0[D:], bu0.reshape(1, -1),
                   Wm1[:D], 2000)
    agg1 = _sc_agg(a1, b1.reshape(E // 2, 2, D), id_flat)
    h2, a2 = _updf(agg1, h1, Wu1[:D], Wu1[D:], bu1.reshape(1, -1),
                   Wm2[:D], 2000)
    agg2 = _sc_agg(a2, b2.reshape(E // 2, 2, D), id_flat)
    pooled = _updpool(batch_idx.reshape(N // 1000, 1, 1000), agg2, h2,
                      Wu2[:D], Wu2[D:], bu2.reshape(1, -1), 1000)
    return _fin(pooled, Wmlp, bmlp.reshape(1, 1))


# speedup vs baseline: 1.5722x; 1.0009x over previous
import functools

import jax
import jax.numpy as jnp
import numpy as np
from jax import lax
from jax.experimental import pallas as pl
from jax.experimental.pallas import tpu as pltpu
from jax.experimental.pallas import tpu_sc as plsc

N = 10000
E = 320000
D = 128
G = 64

NC = 2
NS = 16
CH = 80
EPC = E // NC
EPT = EPC // NS
NCHUNK = EPT // CH
ACC_ROWS = 10048
ZPT = ACC_ROWS // NS
RPT = 624


def _mm_body(x_ref, w_ref, o_ref):
    o_ref[...] = jnp.dot(x_ref[...], w_ref[...],
                         preferred_element_type=jnp.float32)


def _mm(x, w, bn):
    n, k = x.shape
    m = w.shape[1]
    return pl.pallas_call(
        _mm_body,
        grid=(n // bn,),
        in_specs=[pl.BlockSpec((bn, k), lambda i: (i, 0)),
                  pl.BlockSpec((k, m), lambda i: (0, 0))],
        out_specs=pl.BlockSpec((bn, m), lambda i: (i, 0)),
        out_shape=jax.ShapeDtypeStruct((n, m), jnp.float32),
    )(x, w)


def _mmb3_body(x_ref, w0_ref, b0_ref, w1_ref, b1_ref, w2_ref, b2_ref,
               o0_ref, o1_ref, o2_ref):
    x = x_ref[...]
    o0_ref[...] = (jnp.dot(x, w0_ref[...], preferred_element_type=jnp.float32)
                   + b0_ref[...]).astype(jnp.bfloat16)
    o1_ref[...] = (jnp.dot(x, w1_ref[...], preferred_element_type=jnp.float32)
                   + b1_ref[...]).astype(jnp.bfloat16)
    o2_ref[...] = (jnp.dot(x, w2_ref[...], preferred_element_type=jnp.float32)
                   + b2_ref[...]).astype(jnp.bfloat16)


def _mmb3(x, wb, bn):
    n, k = x.shape
    m = wb[0][0].shape[1]
    wspec = pl.BlockSpec((k, m), lambda i: (0, 0))
    bspec = pl.BlockSpec((1, m), lambda i: (0, 0))
    ospec = pl.BlockSpec((bn, m), lambda i: (i, 0))
    oshape = jax.ShapeDtypeStruct((n, m), jnp.bfloat16)
    return pl.pallas_call(
        _mmb3_body,
        grid=(n // bn,),
        in_specs=[pl.BlockSpec((bn, k), lambda i: (i, 0)),
                  wspec, bspec, wspec, bspec, wspec, bspec],
        out_specs=[ospec, ospec, ospec],
        out_shape=[oshape, oshape, oshape],
    )(x, wb[0][0], wb[0][1], wb[1][0], wb[1][1], wb[2][0], wb[2][1])


def _updf_body(a0_ref, a1_ref, h_ref, wa_ref, wh_ref, b_ref, wm_ref,
               o_ref, o2_ref):
    o = jnp.maximum(
        jnp.dot(a0_ref[...] + a1_ref[...], wa_ref[...],
                preferred_element_type=jnp.float32)
        + jnp.dot(h_ref[...], wh_ref[...], preferred_element_type=jnp.float32)
        + b_ref[...], 0.0)
    o_ref[...] = o
    o2_ref[...] = jnp.dot(o, wm_ref[...], preferred_element_type=jnp.float32)


def _updf(agg2, h, wa, wh, b, wm, bn):
    n, k = h.shape
    m = wa.shape[1]
    nb = n // bn
    wspec = pl.BlockSpec((k, m), lambda i: (0, 0))
    return pl.pallas_call(
        _updf_body,
        grid=(nb,),
        in_specs=[pl.BlockSpec((bn, k), lambda i: (i, 0)),
                  pl.BlockSpec((bn, k), lambda i: (i + nb, 0)),
                  pl.BlockSpec((bn, k), lambda i: (i, 0)),
                  wspec, wspec,
                  pl.BlockSpec((1, m), lambda i: (0, 0)),
                  wspec],
        out_specs=[pl.BlockSpec((bn, m), lambda i: (i, 0)),
                   pl.BlockSpec((bn, m), lambda i: (i, 0))],
        out_shape=[jax.ShapeDtypeStruct((n, m), jnp.float32),
                   jax.ShapeDtypeStruct((n, m), jnp.float32)],
    )(agg2, agg2, h, wa, wh, b, wm)


def _updpool_body(idx_ref, a0_ref, a1_ref, h_ref, wa_ref, wh_ref, b_ref,
                  wm_ref, bm_ref, o_ref, pool_ref):
    i = pl.program_id(0)
    nb = pl.num_programs(0)
    y = jnp.maximum(
        jnp.dot(a0_ref[...] + a1_ref[...], wa_ref[...],
                preferred_element_type=jnp.float32)
        + jnp.dot(h_ref[...], wh_ref[...], preferred_element_type=jnp.float32)
        + b_ref[...], 0.0)
    idx = idx_ref[0]
    lab = lax.broadcasted_iota(jnp.int32, (G, idx.shape[1]), 0)
    onehot = (lab == idx).astype(jnp.float32)

    @pl.when(i == 0)
    def _():
        pool_ref[...] = jnp.zeros_like(pool_ref)

    pool_ref[...] += jnp.dot(onehot, y, preferred_element_type=jnp.float32)

    @pl.when(i == nb - 1)
    def _():
        o_ref[...] = jnp.dot(pool_ref[...], wm_ref[...],
                             preferred_element_type=jnp.float32) + bm_ref[...]


def _updpool(batch_idx3, agg2, h, wa, wh, b, wm, bm, bn):
    n, k = h.shape
    m = wa.shape[1]
    nb = n // bn
    wspec = pl.BlockSpec((k, m), lambda i: (0, 0))
    return pl.pallas_call(
        _updpool_body,
        grid=(nb,),
        in_specs=[pl.BlockSpec((1, 1, bn), lambda i: (i, 0, 0)),
                  pl.BlockSpec((bn, k), lambda i: (i, 0)),
                  pl.BlockSpec((bn, k), lambda i: (i + nb, 0)),
                  pl.BlockSpec((bn, k), lambda i: (i, 0)),
                  wspec, wspec,
                  pl.BlockSpec((1, m), lambda i: (0, 0)),
                  pl.BlockSpec((k, 1), lambda i: (0, 0)),
                  pl.BlockSpec((1, 1), lambda i: (0, 0))],
        out_specs=pl.BlockSpec((G, 1), lambda i: (0, 0)),
        out_shape=jax.ShapeDtypeStruct((G, 1), jnp.float32),
        scratch_shapes=[pltpu.VMEM((G, m), jnp.float32)],
    )(batch_idx3, agg2, agg2, h, wa, wh, b, wm, bm)


NBUF = 3
NMAIN = 120
UNR = 8


def _sc_agg_body(a_hbm, b_hbm, id_hbm, out_hbm, acc,
                 iv0, iv1, iv2, dc0, dc1, dc2,
                 ar0, ar1, ar2, br0, br1,
                 ss0, ss1, ss2, sd0, sd1, sd2, sb0, sb1,
                 sg0, sg1, sg2, sx0, sx1, sx2):
    c = lax.axis_index("c")
    s = lax.axis_index("s")
    IV, DC = (iv0, iv1, iv2), (dc0, dc1, dc2)
    AR, BR = (ar0, ar1, ar2), (br0, br1)
    SS, SD, SB = (ss0, ss1, ss2), (sd0, sd1, sd2), (sb0, sb1)
    SG, SX = (sg0, sg1, sg2), (sx0, sx1, sx2)

    zero = jnp.zeros((16,), jnp.float32)

    def zrow_loop(r, carry):
        for k in range(D // 16):
            ar0[r, pl.ds(k * 16, 16)] = zero
        return carry
    lax.fori_loop(0, CH, zrow_loop, 0)

    def zloop64(r, carry):
        pltpu.sync_copy(ar0.at[pl.ds(0, 64)],
                        acc.at[pl.ds(s * ZPT + r * 64, 64)])
        return carry
    lax.fori_loop(0, 9, zloop64, 0)

    def zloop8(r, carry):
        pltpu.sync_copy(ar0.at[pl.ds(0, 8)],
                        acc.at[pl.ds(s * ZPT + 576 + r * 8, 8)])
        return carry
    lax.fori_loop(0, 6, zloop8, 0)
    pltpu.sync_copy(ar0.at[pl.ds(0, 4)], acc.at[pl.ds(s * ZPT + 624, 4)])

    plsc.subcore_barrier()

    def ebase(g):
        return c * EPC + s * EPT + g * CH

    def copy_idx(g, p):
        pltpu.async_copy(id_hbm.at[pl.ds(ebase(g), CH)], IV[p].at[0], SS[p])
        pltpu.async_copy(id_hbm.at[pl.ds(E + ebase(g), CH)],
                         IV[p].at[1], SD[p])

    def copy_b(g, p2):
        pltpu.async_copy(b_hbm.at[pl.ds(ebase(g) // 2, CH // 2)],
                         BR[p2], SB[p2])

    def wait_idx(g, p):
        pltpu.make_async_copy(id_hbm.at[pl.ds(ebase(g), CH)],
                              IV[p].at[0], SS[p]).wait()

    def wait_dst(g, p):
        pltpu.make_async_copy(id_hbm.at[pl.ds(E + ebase(g), CH)],
                              IV[p].at[1], SD[p]).wait()

    def wait_scatter(p):
        pltpu.make_async_copy(AR[p], acc.at[DC[p]], SX[p]).wait()

    def issue_gather(p):
        pltpu.async_copy(a_hbm.at[IV[p].at[0]], AR[p], SG[p])

    def compute(p, p2, nrows):
        @plsc.parallel_loop(0, nrows // 2, step=1, unroll=UNR // 2)
        def _(j2):
            for u in range(2):
                j = j2 * 2 + u
                for k in range(D // 32):
                    v = BR[p2][j2, u, pl.ds(k * 32, 32)].astype(jnp.float32)
                    sl0 = pl.ds(k * 32, 16)
                    sl1 = pl.ds(k * 32 + 16, 16)
                    AR[p][j, sl0] = jnp.maximum(
                        AR[p][j, sl0] + lax.slice(v, (0,), (16,)), 0.0)
                    AR[p][j, sl1] = jnp.maximum(
                        AR[p][j, sl1] + lax.slice(v, (16,), (32,)), 0.0)

    def remap(g, p, nrows):
        wait_dst(g, p)
        for i in range(nrows // 16):
            sl = pl.ds(i * 16, 16)
            DC[p][sl] = IV[p][1, sl]

    copy_idx(0, 0)
    copy_idx(1, 1)
    copy_b(0, 0)
    wait_idx(0, 0)
    issue_gather(0)

    def chunk_body(g, p, p2, q, q2, r, prefetch1, prefetch2, guard):
        if prefetch1:
            wait_idx(g + 1, q)
            if guard:
                @pl.when(g >= 2)
                def _():
                    wait_scatter(q)
            else:
                wait_scatter(q)
            issue_gather(q)
            copy_b(g + 1, q2)
        if prefetch2:
            copy_idx(g + 2, r)
        pltpu.make_async_copy(a_hbm.at[IV[p].at[0]], AR[p], SG[p]).wait()
        pltpu.make_async_copy(b_hbm.at[pl.ds(ebase(g) // 2, CH // 2)],
                              BR[p2], SB[p2]).wait()
        compute(p, p2, CH)
        remap(g, p, CH)
        pltpu.async_copy(AR[p], acc.at[DC[p]], SX[p], add=True)

    def outer_body(o, carry):
        for u in range(6):
            chunk_body(o * 6 + u, u % NBUF, u % 2, (u + 1) % NBUF,
                       (u + 1) % 2, (u + 2) % NBUF, True, True, True)
        return carry
    lax.fori_loop(0, NMAIN // 6, outer_body, 0)

    for g in range(NMAIN, NCHUNK):
        chunk_body(g, g % NBUF, g % 2, (g + 1) % NBUF, (g + 1) % 2,
                   (g + 2) % NBUF, g + 1 < NCHUNK, g + 2 < NCHUNK, False)

    for p in range(NBUF):
        wait_scatter(p)

    plsc.subcore_barrier()

    out0 = c * N + s * RPT
    pltpu.sync_copy(acc.at[pl.ds(s * RPT, RPT)], out_hbm.at[pl.ds(out0, RPT)])

    @pl.when(s == 0)
    def _():
        pltpu.sync_copy(acc.at[pl.ds(NS * RPT, N - NS * RPT)],
                        out_hbm.at[pl.ds(c * N + NS * RPT, N - NS * RPT)])


def _sc_agg(a, b, id_xe):
    mesh = plsc.VectorSubcoreMesh(core_axis_name="c", subcore_axis_name="s")
    f = functools.partial(
        pl.kernel,
        mesh=mesh,
        out_type=jax.ShapeDtypeStruct((NC * N, D), jnp.float32),
        scratch_types=(
            [pltpu.VMEM_SHARED((ACC_ROWS, D), jnp.float32)]
            + [pltpu.VMEM((2, CH), jnp.int32)] * 3
            + [pltpu.VMEM((CH,), jnp.int32)] * 3
            + [pltpu.VMEM((CH, D), jnp.float32)] * 3
            + [pltpu.VMEM((CH // 2, 2, D), jnp.bfloat16)] * 2
            + [pltpu.SemaphoreType.DMA] * 14
        ),
    )(_sc_agg_body)
    return f(a, b, id_xe)


def kernel(H, Xe, id_Xe, batch_idx, Wm0, bm0, Wu0, bu0, Wm1, bm1, Wu1, bu1,
           Wm2, bm2, Wu2, bu2, Wmlp, bmlp):
    b0, b1, b2 = _mmb3(Xe, ((Wm0[D:], bm0.reshape(1, -1)),
                            (Wm1[D:], bm1.reshape(1, -1)),
                            (Wm2[D:], bm2.reshape(1, -1))), 3200)
    a0 = _mm(H, Wm0[:D], 2000)
    id_flat = id_Xe.reshape(-1)
    agg0 = _sc_agg(a0, b0.reshape(E // 2, 2, D), id_flat)
    h1, a1 = _updf(agg0, H, Wu0[:D], Wu0[D:], bu0.reshape(1, -1),
                   Wm1[:D], 2000)
    agg1 = _sc_agg(a1, b1.reshape(E // 2, 2, D), id_flat)
    h2, a2 = _updf(agg1, h1, Wu1[:D], Wu1[D:], bu1.reshape(1, -1),
                   Wm2[:D], 2000)
    agg2 = _sc_agg(a2, b2.reshape(E // 2, 2, D), id_flat)
    return _updpool(batch_idx.reshape(N // 1000, 1, 1000), agg2, h2,
                    Wu2[:D], Wu2[D:], bu2.reshape(1, -1),
                    Wmlp, bmlp.reshape(1, 1), 1000)


# speedup vs baseline: 1.6291x; 1.0362x over previous
import functools

import jax
import jax.numpy as jnp
import numpy as np
from jax import lax
from jax.experimental import pallas as pl
from jax.experimental.pallas import tpu as pltpu
from jax.experimental.pallas import tpu_sc as plsc

N = 10000
E = 320000
D = 128
G = 64

NC = 2
NS = 16
CH = 80
EPC = E // NC
EPT = EPC // NS
NCHUNK = EPT // CH
ACC_ROWS = 10048
ZPT = ACC_ROWS // NS
RPT = 624


def _mm_body(x_ref, w_ref, o_ref):
    o_ref[...] = jnp.dot(x_ref[...], w_ref[...],
                         preferred_element_type=jnp.float32)


def _mm(x, w, bn):
    n, k = x.shape
    m = w.shape[1]
    return pl.pallas_call(
        _mm_body,
        grid=(n // bn,),
        in_specs=[pl.BlockSpec((bn, k), lambda i: (i, 0)),
                  pl.BlockSpec((k, m), lambda i: (0, 0))],
        out_specs=pl.BlockSpec((bn, m), lambda i: (i, 0)),
        out_shape=jax.ShapeDtypeStruct((n, m), jnp.float32),
    )(x, w)


def _mmb3_body(x_ref, w0_ref, b0_ref, w1_ref, b1_ref, w2_ref, b2_ref,
               o0_ref, o1_ref, o2_ref):
    x = x_ref[...]
    o0_ref[...] = (jnp.dot(x, w0_ref[...], preferred_element_type=jnp.float32)
                   + b0_ref[...]).astype(jnp.bfloat16)
    o1_ref[...] = (jnp.dot(x, w1_ref[...], preferred_element_type=jnp.float32)
                   + b1_ref[...]).astype(jnp.bfloat16)
    o2_ref[...] = (jnp.dot(x, w2_ref[...], preferred_element_type=jnp.float32)
                   + b2_ref[...]).astype(jnp.bfloat16)


def _mmb3(x, wb, bn):
    n, k = x.shape
    m = wb[0][0].shape[1]
    wspec = pl.BlockSpec((k, m), lambda i: (0, 0))
    bspec = pl.BlockSpec((1, m), lambda i: (0, 0))
    ospec = pl.BlockSpec((bn, m), lambda i: (i, 0))
    oshape = jax.ShapeDtypeStruct((n, m), jnp.bfloat16)
    return pl.pallas_call(
        _mmb3_body,
        grid=(n // bn,),
        in_specs=[pl.BlockSpec((bn, k), lambda i: (i, 0)),
                  wspec, bspec, wspec, bspec, wspec, bspec],
        out_specs=[ospec, ospec, ospec],
        out_shape=[oshape, oshape, oshape],
    )(x, wb[0][0], wb[0][1], wb[1][0], wb[1][1], wb[2][0], wb[2][1])


def _updf_body(a0_ref, a1_ref, h_ref, wa_ref, wh_ref, b_ref, wm_ref,
               o_ref, o2_ref):
    o = jnp.maximum(
        jnp.dot(a0_ref[...] + a1_ref[...], wa_ref[...],
                preferred_element_type=jnp.float32)
        + jnp.dot(h_ref[...], wh_ref[...], preferred_element_type=jnp.float32)
        + b_ref[...], 0.0)
    o_ref[...] = o
    o2_ref[...] = jnp.dot(o, wm_ref[...], preferred_element_type=jnp.float32)


def _updf(agg2, h, wa, wh, b, wm, bn):
    n, k = h.shape
    m = wa.shape[1]
    nb = n // bn
    wspec = pl.BlockSpec((k, m), lambda i: (0, 0))
    return pl.pallas_call(
        _updf_body,
        grid=(nb,),
        in_specs=[pl.BlockSpec((bn, k), lambda i: (i, 0)),
                  pl.BlockSpec((bn, k), lambda i: (i + nb, 0)),
                  pl.BlockSpec((bn, k), lambda i: (i, 0)),
                  wspec, wspec,
                  pl.BlockSpec((1, m), lambda i: (0, 0)),
                  wspec],
        out_specs=[pl.BlockSpec((bn, m), lambda i: (i, 0)),
                   pl.BlockSpec((bn, m), lambda i: (i, 0))],
        out_shape=[jax.ShapeDtypeStruct((n, m), jnp.float32),
                   jax.ShapeDtypeStruct((n, m), jnp.float32)],
    )(agg2, agg2, h, wa, wh, b, wm)


def _updpool_body(idx_ref, a0_ref, a1_ref, h_ref, wa_ref, wh_ref, b_ref,
                  wm_ref, bm_ref, o_ref, pool_ref):
    i = pl.program_id(0)
    nb = pl.num_programs(0)
    y = jnp.maximum(
        jnp.dot(a0_ref[...] + a1_ref[...], wa_ref[...],
                preferred_element_type=jnp.float32)
        + jnp.dot(h_ref[...], wh_ref[...], preferred_element_type=jnp.float32)
        + b_ref[...], 0.0)
    idx = idx_ref[0]
    lab = lax.broadcasted_iota(jnp.int32, (G, idx.shape[1]), 0)
    onehot = (lab == idx).astype(jnp.float32)

    @pl.when(i == 0)
    def _():
        pool_ref[...] = jnp.zeros_like(pool_ref)

    pool_ref[...] += jnp.dot(onehot, y, preferred_element_type=jnp.float32)

    @pl.when(i == nb - 1)
    def _():
        o_ref[...] = jnp.dot(pool_ref[...], wm_ref[...],
                             preferred_element_type=jnp.float32) + bm_ref[...]


def _updpool(batch_idx3, agg2, h, wa, wh, b, wm, bm, bn):
    n, k = h.shape
    m = wa.shape[1]
    nb = n // bn
    wspec = pl.BlockSpec((k, m), lambda i: (0, 0))
    return pl.pallas_call(
        _updpool_body,
        grid=(nb,),
        in_specs=[pl.BlockSpec((1, 1, bn), lambda i: (i, 0, 0)),
                  pl.BlockSpec((bn, k), lambda i: (i, 0)),
                  pl.BlockSpec((bn, k), lambda i: (i + nb, 0)),
                  pl.BlockSpec((bn, k), lambda i: (i, 0)),
                  wspec, wspec,
                  pl.BlockSpec((1, m), lambda i: (0, 0)),
                  pl.BlockSpec((k, 1), lambda i: (0, 0)),
                  pl.BlockSpec((1, 1), lambda i: (0, 0))],
        out_specs=pl.BlockSpec((G, 1), lambda i: (0, 0)),
        out_shape=jax.ShapeDtypeStruct((G, 1), jnp.float32),
        scratch_shapes=[pltpu.VMEM((G, m), jnp.float32)],
    )(batch_idx3, agg2, agg2, h, wa, wh, b, wm, bm)


NBUF = 3
NMAIN = 123
UNR = 8


def _sc_agg_body(a_hbm, b_hbm, id_hbm, out_hbm, acc,
                 iv0, iv1, iv2, dc0, dc1, dc2,
                 ar0, ar1, ar2, br0, br1, br2,
                 ss0, ss1, ss2, sd0, sd1, sd2, sb0, sb1, sb2,
                 sg0, sg1, sg2, sx0, sx1, sx2):
    c = lax.axis_index("c")
    s = lax.axis_index("s")
    IV, DC = (iv0, iv1, iv2), (dc0, dc1, dc2)
    AR, BR = (ar0, ar1, ar2), (br0, br1, br2)
    SS, SD, SB = (ss0, ss1, ss2), (sd0, sd1, sd2), (sb0, sb1, sb2)
    SG, SX = (sg0, sg1, sg2), (sx0, sx1, sx2)

    zero = jnp.zeros((16,), jnp.float32)

    def zrow_loop(r, carry):
        for k in range(D // 16):
            ar0[r, pl.ds(k * 16, 16)] = zero
        return carry
    lax.fori_loop(0, CH, zrow_loop, 0)

    def zloop64(r, carry):
        pltpu.sync_copy(ar0.at[pl.ds(0, 64)],
                        acc.at[pl.ds(s * ZPT + r * 64, 64)])
        return carry
    lax.fori_loop(0, 9, zloop64, 0)

    def zloop8(r, carry):
        pltpu.sync_copy(ar0.at[pl.ds(0, 8)],
                        acc.at[pl.ds(s * ZPT + 576 + r * 8, 8)])
        return carry
    lax.fori_loop(0, 6, zloop8, 0)
    pltpu.sync_copy(ar0.at[pl.ds(0, 4)], acc.at[pl.ds(s * ZPT + 624, 4)])

    plsc.subcore_barrier()

    def ebase(g):
        return c * EPC + s * EPT + g * CH

    def copy_idx(g, p):
        pltpu.async_copy(id_hbm.at[pl.ds(ebase(g), CH)], IV[p].at[0], SS[p])
        pltpu.async_copy(id_hbm.at[pl.ds(E + ebase(g), CH)],
                         IV[p].at[1], SD[p])

    def copy_b(g, p2):
        pltpu.async_copy(b_hbm.at[pl.ds(ebase(g) // 2, CH // 2)],
                         BR[p2], SB[p2])

    def wait_idx(g, p):
        pltpu.make_async_copy(id_hbm.at[pl.ds(ebase(g), CH)],
                              IV[p].at[0], SS[p]).wait()

    def wait_dst(g, p):
        pltpu.make_async_copy(id_hbm.at[pl.ds(E + ebase(g), CH)],
                              IV[p].at[1], SD[p]).wait()

    def wait_scatter(p):
        pltpu.make_async_copy(AR[p], acc.at[DC[p]], SX[p]).wait()

    def issue_gather(p):
        pltpu.async_copy(a_hbm.at[IV[p].at[0]], AR[p], SG[p])

    def compute(p, p2, nrows):
        @plsc.parallel_loop(0, nrows // 2, step=1, unroll=UNR)
        def _(j2):
            for u in range(2):
                j = j2 * 2 + u
                for k in range(D // 32):
                    v = BR[p2][j2, u, pl.ds(k * 32, 32)].astype(jnp.float32)
                    sl0 = pl.ds(k * 32, 16)
                    sl1 = pl.ds(k * 32 + 16, 16)
                    AR[p][j, sl0] = jnp.maximum(
                        AR[p][j, sl0] + lax.slice(v, (0,), (16,)), 0.0)
                    AR[p][j, sl1] = jnp.maximum(
                        AR[p][j, sl1] + lax.slice(v, (16,), (32,)), 0.0)

    def remap(g, p, nrows):
        wait_dst(g, p)
        for i in range(nrows // 16):
            sl = pl.ds(i * 16, 16)
            DC[p][sl] = IV[p][1, sl]

    copy_idx(0, 0)
    copy_idx(1, 1)
    copy_b(0, 0)
    wait_idx(0, 0)
    issue_gather(0)

    def chunk_body(g, p, q, r, prefetch1, prefetch2, guard):
        if prefetch1:
            wait_idx(g + 1, q)
            if guard:
                @pl.when(g >= 2)
                def _():
                    wait_scatter(q)
            else:
                wait_scatter(q)
            issue_gather(q)
            copy_b(g + 1, q)
        if prefetch2:
            copy_idx(g + 2, r)
        pltpu.make_async_copy(a_hbm.at[IV[p].at[0]], AR[p], SG[p]).wait()
        pltpu.make_async_copy(b_hbm.at[pl.ds(ebase(g) // 2, CH // 2)],
                              BR[p], SB[p]).wait()
        compute(p, p, CH)
        remap(g, p, CH)
        pltpu.async_copy(AR[p], acc.at[DC[p]], SX[p], add=True)

    def outer_body(o, carry):
        for u in range(NBUF):
            chunk_body(o * NBUF + u, u, (u + 1) % NBUF, (u + 2) % NBUF,
                       True, True, True)
        return carry
    lax.fori_loop(0, NMAIN // NBUF, outer_body, 0)

    for g in range(NMAIN, NCHUNK):
        chunk_body(g, g % NBUF, (g + 1) % NBUF, (g + 2) % NBUF,
                   g + 1 < NCHUNK, g + 2 < NCHUNK, False)

    for p in range(NBUF):
        wait_scatter(p)

    plsc.subcore_barrier()

    out0 = c * N + s * RPT
    pltpu.sync_copy(acc.at[pl.ds(s * RPT, RPT)], out_hbm.at[pl.ds(out0, RPT)])

    @pl.when(s == 0)
    def _():
        pltpu.sync_copy(acc.at[pl.ds(NS * RPT, N - NS * RPT)],
                        out_hbm.at[pl.ds(c * N + NS * RPT, N - NS * RPT)])


def _sc_agg(a, b, id_xe):
    mesh = plsc.VectorSubcoreMesh(core_axis_name="c", subcore_axis_name="s")
    f = functools.partial(
        pl.kernel,
        mesh=mesh,
        out_type=jax.ShapeDtypeStruct((NC * N, D), jnp.float32),
        scratch_types=(
            [pltpu.VMEM_SHARED((ACC_ROWS, D), jnp.float32)]
            + [pltpu.VMEM((2, CH), jnp.int32)] * 3
            + [pltpu.VMEM((CH,), jnp.int32)] * 3
            + [pltpu.VMEM((CH, D), jnp.float32)] * 3
            + [pltpu.VMEM((CH // 2, 2, D), jnp.bfloat16)] * 3
            + [pltpu.SemaphoreType.DMA] * 15
        ),
    )(_sc_agg_body)
    return f(a, b, id_xe)


def kernel(H, Xe, id_Xe, batch_idx, Wm0, bm0, Wu0, bu0, Wm1, bm1, Wu1, bu1,
           Wm2, bm2, Wu2, bu2, Wmlp, bmlp):
    b0, b1, b2 = _mmb3(Xe, ((Wm0[D:], bm0.reshape(1, -1)),
                            (Wm1[D:], bm1.reshape(1, -1)),
                            (Wm2[D:], bm2.reshape(1, -1))), 3200)
    a0 = _mm(H, Wm0[:D], 2000)
    id_flat = id_Xe.reshape(-1)
    agg0 = _sc_agg(a0, b0.reshape(E // 2, 2, D), id_flat)
    h1, a1 = _updf(agg0, H, Wu0[:D], Wu0[D:], bu0.reshape(1, -1),
                   Wm1[:D], 2000)
    agg1 = _sc_agg(a1, b1.reshape(E // 2, 2, D), id_flat)
    h2, a2 = _updf(agg1, h1, Wu1[:D], Wu1[D:], bu1.reshape(1, -1),
                   Wm2[:D], 2000)
    agg2 = _sc_agg(a2, b2.reshape(E // 2, 2, D), id_flat)
    return _updpool(batch_idx.reshape(N // 1000, 1, 1000), agg2, h2,
                    Wu2[:D], Wu2[D:], bu2.reshape(1, -1),
                    Wmlp, bmlp.reshape(1, 1), 1000)
